# baseline (speedup 1.0000x reference)
import functools

import jax
import jax.numpy as jnp
from jax import lax
from jax.experimental import pallas as pl
from jax.experimental.pallas import tpu as pltpu
from jax.experimental.pallas import tpu_sc as plsc

N = 10000
E = 320000
DIN = 128
D = 64
NCLS = 10

NC = 2
NS = 16
NW = NC * NS
EPT = E // NW
B = 80
NBLK = EPT // B

KB = 4

R = 2000
NG = N // R


def _shuffle_bf16(x):
    xb = x.astype(jnp.bfloat16).reshape(N, 2, 2, 16)
    return jnp.transpose(xb, (0, 1, 3, 2)).reshape(N, D)

_mesh = plsc.VectorSubcoreMesh(
    core_axis_name="c", subcore_axis_name="s", num_cores=NC, num_subcores=NS)
_sc_params = pltpu.CompilerParams(use_tc_tiling_on_sc=False,
                                  needs_layout_passes=False)


def _zero_vec(buf, nv):
    def zb(t, carry):
        buf[pl.ds(t * 16, 16)] = jnp.zeros((16,), jnp.float32)
        return carry
    lax.fori_loop(0, nv, zb, 0)


def _deg_body(dst3_hbm, out_hbm, dst_idx, ones_v, buf1, deg_sh, sem):
    c = lax.axis_index("c")
    s = lax.axis_index("s")
    chunk = c * NS + s
    pltpu.sync_copy(dst3_hbm.at[chunk], dst_idx)
    for k in range(B // 16):
        ones_v[pl.ds(k * 16, 16)] = jnp.ones((16,), jnp.float32)
    _zero_vec(buf1, 63)

    @pl.when(s < 10)
    def _():
        pltpu.sync_copy(buf1.at[pl.ds(0, 1000)],
                        deg_sh.at[pl.ds(s * 1000, 1000)])

    plsc.subcore_barrier()

    def blk(j, carry):
        pltpu.sync_copy(ones_v, deg_sh.at[dst_idx.at[j]], add=True)
        return carry

    lax.fori_loop(0, NBLK, blk, 0)
    plsc.subcore_barrier()

    @pl.when(s < 10)
    def _():
        pltpu.sync_copy(deg_sh.at[pl.ds(s * 1000, 1000)],
                        buf1.at[pl.ds(0, 1000)])
        pltpu.sync_copy(buf1.at[pl.ds(0, 1000)],
                        out_hbm.at[pl.ds(c * N + s * 1000, 1000)])


def _sc_deg(dst3):
    return pl.kernel(
        _deg_body,
        out_type=jax.ShapeDtypeStruct((NC * N,), jnp.float32),
        mesh=_mesh,
        compiler_params=_sc_params,
        scratch_types=[
            pltpu.VMEM((NBLK, B), jnp.int32),
            pltpu.VMEM((B,), jnp.float32),
            pltpu.VMEM((1008,), jnp.float32),
            pltpu.VMEM_SHARED((N,), jnp.float32),
            pltpu.SemaphoreType.DMA,
        ],
    )(dst3)


def _zero_rows(buf, nrows):
    def zb(r, carry):
        for t in range(D // 16):
            buf[r, pl.ds(t * 16, 16)] = jnp.zeros((16,), jnp.float32)
        return carry
    lax.fori_loop(0, nrows, zb, 0)


def _acc_init(acc_sh, bounce, s):
    _zero_rows(bounce, 125)

    def zc(ch, carry):
        pltpu.sync_copy(bounce, acc_sh.at[pl.ds(s * 625 + ch * 125, 125)])
        return carry

    lax.fori_loop(0, 5, zc, 0)


def _acc_out(acc_sh, bounce, out_hbm, c, s):
    def oc(ch, carry):
        r0 = s * 625 + ch * 125
        pltpu.sync_copy(acc_sh.at[pl.ds(r0, 125)], bounce)
        pltpu.sync_copy(bounce, out_hbm.at[c, pl.ds(r0, 125)])
        return carry

    lax.fori_loop(0, 5, oc, 0)


def _unpack_row(rows_bf, rowsf, b, i, bw=None):
    for c2 in range(2):
        v = plsc.bitcast(rows_bf[b, i, pl.ds(c2 * 32, 32)], jnp.int32)
        lo = plsc.bitcast(jnp.left_shift(v, 16), jnp.float32)
        hi = plsc.bitcast(jnp.bitwise_and(v, jnp.int32(-65536)), jnp.float32)
        if bw is not None:
            lo = lo * bw
            hi = hi * bw
        rowsf[b, i, pl.ds(c2 * 32, 16)] = lo
        rowsf[b, i, pl.ds(c2 * 32 + 16, 16)] = hi


def _gcn_body(g_hbm, src3_hbm, dst3_hbm, out_hbm,
              src_idx, dst_idx, rows_bf, rowsf, big_buf, acc_sh,
              gsems, ssems):
    c = lax.axis_index("c")
    s = lax.axis_index("s")
    chunk = c * NS + s
    pltpu.sync_copy(src3_hbm.at[chunk], src_idx)
    pltpu.sync_copy(dst3_hbm.at[chunk], dst_idx)
    _acc_init(acc_sh, big_buf, s)
    plsc.subcore_barrier()
    pltpu.async_copy(g_hbm.at[src_idx.at[0]], rows_bf.at[0], gsems.at[0])
    pltpu.async_copy(g_hbm.at[src_idx.at[1]], rows_bf.at[1], gsems.at[1])

    def blk(j, carry):
        b = lax.rem(j, KB)
        pltpu.make_async_copy(g_hbm.at[src_idx.at[j]], rows_bf.at[b],
                              gsems.at[b]).wait()

        @plsc.parallel_loop(0, B, unroll=4)
        def _(i):
            _unpack_row(rows_bf, rowsf, b, i)

        pltpu.async_copy(rowsf.at[b], acc_sh.at[dst_idx.at[j]], ssems.at[b],
                         add=True)

        @pl.when(j + 2 < NBLK)
        def _():
            b2 = lax.rem(j + 2, KB)

            @pl.when(j >= 2)
            def _():
                pltpu.make_async_copy(rowsf.at[b2],
                                      acc_sh.at[dst_idx.at[j - 2]],
                                      ssems.at[b2]).wait()

            pltpu.async_copy(g_hbm.at[src_idx.at[j + 2]], rows_bf.at[b2],
                             gsems.at[b2])

        return carry

    lax.fori_loop(0, NBLK, blk, 0)

    def drain(t, carry):
        jj = NBLK - KB + t
        bb = lax.rem(jj, KB)
        pltpu.make_async_copy(rowsf.at[bb], acc_sh.at[dst_idx.at[jj]],
                              ssems.at[bb]).wait()
        return carry

    lax.fori_loop(0, KB, drain, 0)
    plsc.subcore_barrier()
    _acc_out(acc_sh, big_buf, out_hbm, c, s)


def _sc_gcn(g, src3, dst3):
    return pl.kernel(
        _gcn_body,
        out_type=jax.ShapeDtypeStruct((NC, N, D), jnp.float32),
        mesh=_mesh,
        compiler_params=_sc_params,
        scratch_types=[
            pltpu.VMEM((NBLK, B), jnp.int32),
            pltpu.VMEM((NBLK, B), jnp.int32),
            pltpu.VMEM((KB, B, D), jnp.bfloat16),
            pltpu.VMEM((KB, B, D), jnp.float32),
            pltpu.VMEM((125, D), jnp.float32),
            pltpu.VMEM_SHARED((N, D), jnp.float32),
            pltpu.SemaphoreType.DMA((KB,)),
            pltpu.SemaphoreType.DMA((KB,)),
        ],
    )(g, src3, dst3)


def _gat_body(hh_hbm, p_hbm, q_hbm, pmax_hbm, src3_hbm, dst3_hbm,
              acc_out, s_out,
              src_idx, dst_idx, rows_bf, rowsf, w_buf, p_v, q_v, pm_v, buf1,
              big_buf, acc_sh, s_sh, gsems, ssems):
    c = lax.axis_index("c")
    s = lax.axis_index("s")
    chunk = c * NS + s
    pltpu.sync_copy(src3_hbm.at[chunk], src_idx)
    pltpu.sync_copy(dst3_hbm.at[chunk], dst_idx)
    pltpu.sync_copy(p_hbm, p_v)
    pltpu.sync_copy(q_hbm, q_v)
    pltpu.sync_copy(pmax_hbm, pm_v)
    _acc_init(acc_sh, big_buf, s)
    _zero_vec(buf1, 63)

    @pl.when(s < 10)
    def _():
        pltpu.sync_copy(buf1.at[pl.ds(0, 1000)],
                        s_sh.at[pl.ds(s * 1000, 1000)])

    plsc.subcore_barrier()
    pmv = pm_v[...]
    pltpu.async_copy(hh_hbm.at[src_idx.at[0]], rows_bf.at[0], gsems.at[0])
    pltpu.async_copy(hh_hbm.at[src_idx.at[1]], rows_bf.at[1], gsems.at[1])

    def blk(j, carry):
        b = lax.rem(j, KB)
        for k in range(B // 16):
            si = src_idx[j, pl.ds(k * 16, 16)]
            di = dst_idx[j, pl.ds(k * 16, 16)]
            pv = plsc.load_gather(p_v, [si])
            qv = plsc.load_gather(q_v, [di])
            z = pv + qv
            e = jnp.maximum(z, 0.2 * z)
            zu = pmv + qv
            u = jnp.maximum(zu, 0.2 * zu)
            w_buf[pl.ds(k * 16, 16)] = jnp.exp(e - u)

        pltpu.make_async_copy(hh_hbm.at[src_idx.at[j]], rows_bf.at[b],
                              gsems.at[b]).wait()

        @plsc.parallel_loop(0, B, unroll=4)
        def _(i):
            bw = plsc.load_gather(w_buf, [jnp.full((16,), i, jnp.int32)])
            _unpack_row(rows_bf, rowsf, b, i, bw)

        pltpu.async_copy(rowsf.at[b], acc_sh.at[dst_idx.at[j]], ssems.at[b],
                         add=True)
        pltpu.sync_copy(w_buf, s_sh.at[dst_idx.at[j]], add=True)

        @pl.when(j + 2 < NBLK)
        def _():
            b2 = lax.rem(j + 2, KB)

            @pl.when(j >= 2)
            def _():
                pltpu.make_async_copy(rowsf.at[b2],
                                      acc_sh.at[dst_idx.at[j - 2]],
                                      ssems.at[b2]).wait()

            pltpu.async_copy(hh_hbm.at[src_idx.at[j + 2]], rows_bf.at[b2],
                             gsems.at[b2])

        return carry

    lax.fori_loop(0, NBLK, blk, 0)

    def drain(t, carry):
        jj = NBLK - KB + t
        bb = lax.rem(jj, KB)
        pltpu.make_async_copy(rowsf.at[bb], acc_sh.at[dst_idx.at[jj]],
                              ssems.at[bb]).wait()
        return carry

    lax.fori_loop(0, KB, drain, 0)
    plsc.subcore_barrier()
    _acc_out(acc_sh, big_buf, acc_out, c, s)

    @pl.when(s < 10)
    def _():
        pltpu.sync_copy(s_sh.at[pl.ds(s * 1000, 1000)],
                        buf1.at[pl.ds(0, 1000)])
        pltpu.sync_copy(buf1.at[pl.ds(0, 1000)],
                        s_out.at[pl.ds(c * N + s * 1000, 1000)])


def _sc_gat(hh, p, q, pmax, src3, dst3):
    return pl.kernel(
        _gat_body,
        out_type=(jax.ShapeDtypeStruct((NC, N, D), jnp.float32),
                  jax.ShapeDtypeStruct((NC * N,), jnp.float32)),
        mesh=_mesh,
        compiler_params=_sc_params,
        scratch_types=[
            pltpu.VMEM((NBLK, B), jnp.int32),
            pltpu.VMEM((NBLK, B), jnp.int32),
            pltpu.VMEM((KB, B, D), jnp.bfloat16),
            pltpu.VMEM((KB, B, D), jnp.float32),
            pltpu.VMEM((B,), jnp.float32),
            pltpu.VMEM((N,), jnp.float32),
            pltpu.VMEM((N,), jnp.float32),
            pltpu.VMEM((16,), jnp.float32),
            pltpu.VMEM((1008,), jnp.float32),
            pltpu.VMEM((125, D), jnp.float32),
            pltpu.VMEM_SHARED((N, D), jnp.float32),
            pltpu.VMEM_SHARED((N,), jnp.float32),
            pltpu.SemaphoreType.DMA((KB,)),
            pltpu.SemaphoreType.DMA((KB,)),
        ],
    )(hh, p, q, pmax, src3, dst3)


def _tc1_body(x_ref, w1_ref, da_ref, db_ref, g1_ref, dinv_ref):
    deg = da_ref[...] + db_ref[...] + 1.0
    dv = lax.rsqrt(deg)
    g1_ref[...] = jnp.dot(x_ref[...], w1_ref[...],
                          preferred_element_type=jnp.float32) * dv
    dinv_ref[...] = dv


def _tc1(x, W1, dA, dB):
    return pl.pallas_call(
        _tc1_body,
        grid=(NG,),
        in_specs=[
            pl.BlockSpec((R, DIN), lambda i: (i, 0)),
            pl.BlockSpec((DIN, D), lambda i: (0, 0)),
            pl.BlockSpec((R, 1), lambda i: (i, 0)),
            pl.BlockSpec((R, 1), lambda i: (i, 0)),
        ],
        out_specs=[
            pl.BlockSpec((R, D), lambda i: (i, 0)),
            pl.BlockSpec((R, 1), lambda i: (i, 0)),
        ],
        out_shape=[
            jax.ShapeDtypeStruct((N, D), jnp.float32),
            jax.ShapeDtypeStruct((N, 1), jnp.float32),
        ],
    )(x, W1, dA, dB)


def _tc2_body(accp_ref, g1_ref, dinv_ref, b1_ref, w2_ref, as_ref, ad_ref,
              hh_ref, p_ref, q_ref, pmax_ref):
    i = pl.program_id(0)
    acc = accp_ref[0] + accp_ref[1] + g1_ref[...]
    h1 = jnp.maximum(dinv_ref[...] * acc + b1_ref[...], 0.0)
    hh = jnp.dot(h1, w2_ref[...], preferred_element_type=jnp.float32)
    hh_ref[...] = hh
    p = jnp.dot(hh, as_ref[...], preferred_element_type=jnp.float32)
    q = jnp.dot(hh, ad_ref[...], preferred_element_type=jnp.float32)
    p_ref[...] = p
    q_ref[...] = q
    pb = jnp.max(p, axis=(0, 1), keepdims=True)

    @pl.when(i == 0)
    def _():
        pmax_ref[...] = pb

    @pl.when(i > 0)
    def _():
        pmax_ref[...] = jnp.maximum(pmax_ref[...], pb)


def _tc2(accP, g1, dinv, b1, W2, aS, aD):
    return pl.pallas_call(
        _tc2_body,
        grid=(NG,),
        in_specs=[
            pl.BlockSpec((NC, R, D), lambda i: (0, i, 0)),
            pl.BlockSpec((R, D), lambda i: (i, 0)),
            pl.BlockSpec((R, 1), lambda i: (i, 0)),
            pl.BlockSpec((1, D), lambda i: (0, 0)),
            pl.BlockSpec((D, D), lambda i: (0, 0)),
            pl.BlockSpec((D, 1), lambda i: (0, 0)),
            pl.BlockSpec((D, 1), lambda i: (0, 0)),
        ],
        out_specs=[
            pl.BlockSpec((R, D), lambda i: (i, 0)),
            pl.BlockSpec((R, 1), lambda i: (i, 0)),
            pl.BlockSpec((R, 1), lambda i: (i, 0)),
            pl.BlockSpec((1, 1), lambda i: (0, 0)),
        ],
        out_shape=[
            jax.ShapeDtypeStruct((N, D), jnp.float32),
            jax.ShapeDtypeStruct((N, 1), jnp.float32),
            jax.ShapeDtypeStruct((N, 1), jnp.float32),
            jax.ShapeDtypeStruct((1, 1), jnp.float32),
        ],
    )(accP, g1, dinv, b1, W2, aS, aD)


def _tc3_body(accp_ref, sp_ref, hh_ref, p_ref, q_ref, pmax_ref, dinv_ref,
              b2_ref, w3_ref, g3_ref):
    p = p_ref[...]
    q = q_ref[...]
    z = p + q
    e_self = jnp.maximum(z, 0.2 * z)
    zu = pmax_ref[0, 0] + q
    u = jnp.maximum(zu, 0.2 * zu)
    w_self = jnp.exp(e_self - u)
    den = sp_ref[0] + sp_ref[1] + w_self + 1e-16
    num = accp_ref[0] + accp_ref[1] + w_self * hh_ref[...]
    h2 = jnp.maximum(num / den + b2_ref[...], 0.0)
    g3_ref[...] = jnp.dot(h2, w3_ref[...],
                          preferred_element_type=jnp.float32) * dinv_ref[...]


def _tc3(accP, sP, hh, p, q, pmax, dinv, b2, W3):
    return pl.pallas_call(
        _tc3_body,
        grid=(NG,),
        in_specs=[
            pl.BlockSpec((NC, R, D), lambda i: (0, i, 0)),
            pl.BlockSpec((NC, R, 1), lambda i: (0, i, 0)),
            pl.BlockSpec((R, D), lambda i: (i, 0)),
            pl.BlockSpec((R, 1), lambda i: (i, 0)),
            pl.BlockSpec((R, 1), lambda i: (i, 0)),
            pl.BlockSpec((1, 1), lambda i: (0, 0)),
            pl.BlockSpec((R, 1), lambda i: (i, 0)),
            pl.BlockSpec((1, D), lambda i: (0, 0)),
            pl.BlockSpec((D, D), lambda i: (0, 0)),
        ],
        out_specs=[pl.BlockSpec((R, D), lambda i: (i, 0))],
        out_shape=[jax.ShapeDtypeStruct((N, D), jnp.float32)],
    )(accP, sP, hh, p, q, pmax, dinv, b2, W3)


def _tc4_body(accp_ref, g3_ref, dinv_ref, b3_ref, wo_ref, bo_ref,
              wb1_ref, bb1_ref, wb2_ref, bb2_ref,
              opt_ref, bt_ref, ge_ref):
    i = pl.program_id(0)
    acc = accp_ref[0] + accp_ref[1] + g3_ref[...]
    h3 = jnp.maximum(dinv_ref[...] * acc + b3_ref[...], 0.0)
    opt_ref[...] = jnp.dot(h3, wo_ref[...],
                           preferred_element_type=jnp.float32) + bo_ref[...]
    t = jnp.maximum(jnp.dot(h3, wb1_ref[...],
                            preferred_element_type=jnp.float32) + bb1_ref[...],
                    0.0)
    bt_ref[...] = jax.nn.sigmoid(
        jnp.dot(t, wb2_ref[...], preferred_element_type=jnp.float32)
        + bb2_ref[...])
    tot = jnp.sum(h3, axis=0, keepdims=True)

    @pl.when(i == 0)
    def _():
        ge_ref[...] = tot

    @pl.when(i > 0)
    def _():
        ge_ref[...] = ge_ref[...] + tot

    @pl.when(i == NG - 1)
    def _():
        ge_ref[...] = ge_ref[...] * (1.0 / N)


def _tc4(accP, g3, dinv, b3, Wo, bo, Wb1, bb1, Wb2, bb2):
    return pl.pallas_call(
        _tc4_body,
        grid=(NG,),
        in_specs=[
            pl.BlockSpec((NC, R, D), lambda i: (0, i, 0)),
            pl.BlockSpec((R, D), lambda i: (i, 0)),
            pl.BlockSpec((R, 1), lambda i: (i, 0)),
            pl.BlockSpec((1, D), lambda i: (0, 0)),
            pl.BlockSpec((D, NCLS), lambda i: (0, 0)),
            pl.BlockSpec((1, NCLS), lambda i: (0, 0)),
            pl.BlockSpec((D, 32), lambda i: (0, 0)),
            pl.BlockSpec((1, 32), lambda i: (0, 0)),
            pl.BlockSpec((32, 1), lambda i: (0, 0)),
            pl.BlockSpec((1, 1), lambda i: (0, 0)),
        ],
        out_specs=[
            pl.BlockSpec((R, NCLS), lambda i: (i, 0)),
            pl.BlockSpec((R, 1), lambda i: (i, 0)),
            pl.BlockSpec((1, D), lambda i: (0, 0)),
        ],
        out_shape=[
            jax.ShapeDtypeStruct((N, NCLS), jnp.float32),
            jax.ShapeDtypeStruct((N, 1), jnp.float32),
            jax.ShapeDtypeStruct((1, D), jnp.float32),
        ],
    )(accP, g3, dinv, b3, Wo, bo, Wb1, bb1, Wb2, bb2)


def kernel(x, edge_index, W1, b1, W2, a_src, a_dst, b2, W3, b3, Wo, bo,
           Wb1, bb1, Wb2, bb2):
    src3 = edge_index[0].reshape(NW, NBLK, B)
    dst3 = edge_index[1].reshape(NW, NBLK, B)

    degP = _sc_deg(dst3).reshape(NC, N)
    dA = degP[0].reshape(N, 1)
    dB = degP[1].reshape(N, 1)

    g1, dinv = _tc1(x, W1, dA, dB)
    acc1 = _sc_gcn(_shuffle_bf16(g1), src3, dst3)
    hh, p, q, pmax = _tc2(acc1, g1, dinv, b1.reshape(1, D), W2,
                          a_src.reshape(D, 1), a_dst.reshape(D, 1))

    pmax16 = jnp.broadcast_to(pmax.reshape(1), (16,))
    acc2, s2 = _sc_gat(_shuffle_bf16(hh), p.reshape(N), q.reshape(N), pmax16,
                       src3, dst3)
    g3 = _tc3(acc2, s2.reshape(NC, N, 1), hh, p, q, pmax, dinv,
              b2.reshape(1, D), W3)[0]

    acc3 = _sc_gcn(_shuffle_bf16(g3), src3, dst3)
    opt, bt, ge = _tc4(acc3, g3, dinv, b3.reshape(1, D), Wo,
                       bo.reshape(1, NCLS), Wb1, bb1.reshape(1, 32),
                       Wb2, bb2.reshape(1, 1))
    return opt, bt, ge.reshape(D)


# speedup vs baseline: 1.0390x; 1.0390x over previous
import functools

import jax
import jax.numpy as jnp
import numpy as np
from jax import lax
from jax.experimental import pallas as pl
from jax.experimental.pallas import tpu as pltpu
from jax.experimental.pallas import tpu_sc as plsc

N = 10000
E = 320000
DIN = 128
D = 64
NCLS = 10

NC = 2
NS = 16
NW = NC * NS
EPT = E // NW
B = 80
NBLK = EPT // B

KB = 4

R = 2000
NG = N // R


_TAU = np.empty(D, np.int32)
for _c in range(2):
    for _k in range(16):
        _TAU[_c * 32 + _k] = _c * 32 + 2 * _k
        _TAU[_c * 32 + 16 + _k] = _c * 32 + 2 * _k + 1
_INV_TAU = np.argsort(_TAU)

_mesh = plsc.VectorSubcoreMesh(
    core_axis_name="c", subcore_axis_name="s", num_cores=NC, num_subcores=NS)
_sc_params = pltpu.CompilerParams(use_tc_tiling_on_sc=False,
                                  needs_layout_passes=False)


def _zero_vec(buf, nv):
    def zb(t, carry):
        buf[pl.ds(t * 16, 16)] = jnp.zeros((16,), jnp.float32)
        return carry
    lax.fori_loop(0, nv, zb, 0)


def _deg_body(dst3_hbm, out_hbm, dst_idx, ones_v, buf1, deg_sh, sem):
    c = lax.axis_index("c")
    s = lax.axis_index("s")
    chunk = c * NS + s
    pltpu.sync_copy(dst3_hbm.at[chunk], dst_idx)
    for k in range(B // 16):
        ones_v[pl.ds(k * 16, 16)] = jnp.ones((16,), jnp.float32)
    _zero_vec(buf1, 63)

    @pl.when(s < 10)
    def _():
        pltpu.sync_copy(buf1.at[pl.ds(0, 1000)],
                        deg_sh.at[pl.ds(s * 1000, 1000)])

    plsc.subcore_barrier()

    def blk(j, carry):
        pltpu.sync_copy(ones_v, deg_sh.at[dst_idx.at[j]], add=True)
        return carry

    lax.fori_loop(0, NBLK, blk, 0)
    plsc.subcore_barrier()

    @pl.when(s < 10)
    def _():
        pltpu.sync_copy(deg_sh.at[pl.ds(s * 1000, 1000)],
                        buf1.at[pl.ds(0, 1000)])
        pltpu.sync_copy(buf1.at[pl.ds(0, 1000)],
                        out_hbm.at[pl.ds(c * N + s * 1000, 1000)])


def _sc_deg(dst3):
    return pl.kernel(
        _deg_body,
        out_type=jax.ShapeDtypeStruct((NC * N,), jnp.float32),
        mesh=_mesh,
        compiler_params=_sc_params,
        scratch_types=[
            pltpu.VMEM((NBLK, B), jnp.int32),
            pltpu.VMEM((B,), jnp.float32),
            pltpu.VMEM((1008,), jnp.float32),
            pltpu.VMEM_SHARED((N,), jnp.float32),
            pltpu.SemaphoreType.DMA,
        ],
    )(dst3)


def _zero_rows(buf, nrows):
    def zb(r, carry):
        for t in range(D // 16):
            buf[r, pl.ds(t * 16, 16)] = jnp.zeros((16,), jnp.float32)
        return carry
    lax.fori_loop(0, nrows, zb, 0)


def _acc_init(acc_sh, bounce, s):
    _zero_rows(bounce, 125)

    def zc(ch, carry):
        pltpu.sync_copy(bounce, acc_sh.at[pl.ds(s * 625 + ch * 125, 125)])
        return carry

    lax.fori_loop(0, 5, zc, 0)


def _acc_out(acc_sh, bounce, out_hbm, c, s):
    def oc(ch, carry):
        r0 = s * 625 + ch * 125
        pltpu.sync_copy(acc_sh.at[pl.ds(r0, 125)], bounce)
        pltpu.sync_copy(bounce, out_hbm.at[c, pl.ds(r0, 125)])
        return carry

    lax.fori_loop(0, 5, oc, 0)


def _unpack_row(rows_bf, rowsf, b, i, bw=None):
    for c2 in range(2):
        v = plsc.bitcast(rows_bf[b, i, pl.ds(c2 * 32, 32)], jnp.int32)
        lo = plsc.bitcast(jnp.left_shift(v, 16), jnp.float32)
        hi = plsc.bitcast(jnp.bitwise_and(v, jnp.int32(-65536)), jnp.float32)
        if bw is not None:
            lo = lo * bw
            hi = hi * bw
        rowsf[b, i, pl.ds(c2 * 32, 16)] = lo
        rowsf[b, i, pl.ds(c2 * 32 + 16, 16)] = hi


def _gcn_body(g_hbm, src3_hbm, dst3_hbm, out_hbm,
              src_idx, dst_idx, rows_bf, rowsf, big_buf, acc_sh,
              gsems, ssems):
    c = lax.axis_index("c")
    s = lax.axis_index("s")
    chunk = c * NS + s
    pltpu.sync_copy(src3_hbm.at[chunk], src_idx)
    pltpu.sync_copy(dst3_hbm.at[chunk], dst_idx)
    _acc_init(acc_sh, big_buf, s)
    plsc.subcore_barrier()
    pltpu.async_copy(g_hbm.at[src_idx.at[0]], rows_bf.at[0], gsems.at[0])
    pltpu.async_copy(g_hbm.at[src_idx.at[1]], rows_bf.at[1], gsems.at[1])

    def blk(j, carry):
        b = lax.rem(j, KB)
        pltpu.make_async_copy(g_hbm.at[src_idx.at[j]], rows_bf.at[b],
                              gsems.at[b]).wait()

        @plsc.parallel_loop(0, B, unroll=4)
        def _(i):
            _unpack_row(rows_bf, rowsf, b, i)

        pltpu.async_copy(rowsf.at[b], acc_sh.at[dst_idx.at[j]], ssems.at[b],
                         add=True)

        @pl.when(j + 2 < NBLK)
        def _():
            b2 = lax.rem(j + 2, KB)

            @pl.when(j >= 2)
            def _():
                pltpu.make_async_copy(rowsf.at[b2],
                                      acc_sh.at[dst_idx.at[j - 2]],
                                      ssems.at[b2]).wait()

            pltpu.async_copy(g_hbm.at[src_idx.at[j + 2]], rows_bf.at[b2],
                             gsems.at[b2])

        return carry

    lax.fori_loop(0, NBLK, blk, 0)

    def drain(t, carry):
        jj = NBLK - KB + t
        bb = lax.rem(jj, KB)
        pltpu.make_async_copy(rowsf.at[bb], acc_sh.at[dst_idx.at[jj]],
                              ssems.at[bb]).wait()
        return carry

    lax.fori_loop(0, KB, drain, 0)
    plsc.subcore_barrier()
    _acc_out(acc_sh, big_buf, out_hbm, c, s)


def _sc_gcn(g, src3, dst3):
    return pl.kernel(
        _gcn_body,
        out_type=jax.ShapeDtypeStruct((NC, N, D), jnp.float32),
        mesh=_mesh,
        compiler_params=_sc_params,
        scratch_types=[
            pltpu.VMEM((NBLK, B), jnp.int32),
            pltpu.VMEM((NBLK, B), jnp.int32),
            pltpu.VMEM((KB, B, D), jnp.bfloat16),
            pltpu.VMEM((KB, B, D), jnp.float32),
            pltpu.VMEM((125, D), jnp.float32),
            pltpu.VMEM_SHARED((N, D), jnp.float32),
            pltpu.SemaphoreType.DMA((KB,)),
            pltpu.SemaphoreType.DMA((KB,)),
        ],
    )(g, src3, dst3)


def _gat_body(hh_hbm, p_hbm, q_hbm, pmax_hbm, src3_hbm, dst3_hbm,
              acc_out, s_out,
              src_idx, dst_idx, rows_bf, rowsf, w_buf, p_v, q_v, pm_v, buf1,
              big_buf, acc_sh, s_sh, gsems, ssems):
    c = lax.axis_index("c")
    s = lax.axis_index("s")
    chunk = c * NS + s
    pltpu.sync_copy(src3_hbm.at[chunk], src_idx)
    pltpu.sync_copy(dst3_hbm.at[chunk], dst_idx)
    pltpu.sync_copy(p_hbm, p_v)
    pltpu.sync_copy(q_hbm, q_v)
    pltpu.sync_copy(pmax_hbm, pm_v)
    _acc_init(acc_sh, big_buf, s)
    _zero_vec(buf1, 63)

    @pl.when(s < 10)
    def _():
        pltpu.sync_copy(buf1.at[pl.ds(0, 1000)],
                        s_sh.at[pl.ds(s * 1000, 1000)])

    plsc.subcore_barrier()
    pmv = pm_v[...]
    pltpu.async_copy(hh_hbm.at[src_idx.at[0]], rows_bf.at[0], gsems.at[0])
    pltpu.async_copy(hh_hbm.at[src_idx.at[1]], rows_bf.at[1], gsems.at[1])

    def blk(j, carry):
        b = lax.rem(j, KB)
        for k in range(B // 16):
            si = src_idx[j, pl.ds(k * 16, 16)]
            di = dst_idx[j, pl.ds(k * 16, 16)]
            pv = plsc.load_gather(p_v, [si])
            qv = plsc.load_gather(q_v, [di])
            z = pv + qv
            e = jnp.maximum(z, 0.2 * z)
            zu = pmv + qv
            u = jnp.maximum(zu, 0.2 * zu)
            w_buf[pl.ds(k * 16, 16)] = jnp.exp(e - u)

        pltpu.make_async_copy(hh_hbm.at[src_idx.at[j]], rows_bf.at[b],
                              gsems.at[b]).wait()

        @plsc.parallel_loop(0, B, unroll=4)
        def _(i):
            bw = plsc.load_gather(w_buf, [jnp.full((16,), i, jnp.int32)])
            _unpack_row(rows_bf, rowsf, b, i, bw)

        pltpu.async_copy(rowsf.at[b], acc_sh.at[dst_idx.at[j]], ssems.at[b],
                         add=True)
        pltpu.sync_copy(w_buf, s_sh.at[dst_idx.at[j]], add=True)

        @pl.when(j + 2 < NBLK)
        def _():
            b2 = lax.rem(j + 2, KB)

            @pl.when(j >= 2)
            def _():
                pltpu.make_async_copy(rowsf.at[b2],
                                      acc_sh.at[dst_idx.at[j - 2]],
                                      ssems.at[b2]).wait()

            pltpu.async_copy(hh_hbm.at[src_idx.at[j + 2]], rows_bf.at[b2],
                             gsems.at[b2])

        return carry

    lax.fori_loop(0, NBLK, blk, 0)

    def drain(t, carry):
        jj = NBLK - KB + t
        bb = lax.rem(jj, KB)
        pltpu.make_async_copy(rowsf.at[bb], acc_sh.at[dst_idx.at[jj]],
                              ssems.at[bb]).wait()
        return carry

    lax.fori_loop(0, KB, drain, 0)
    plsc.subcore_barrier()
    _acc_out(acc_sh, big_buf, acc_out, c, s)

    @pl.when(s < 10)
    def _():
        pltpu.sync_copy(s_sh.at[pl.ds(s * 1000, 1000)],
                        buf1.at[pl.ds(0, 1000)])
        pltpu.sync_copy(buf1.at[pl.ds(0, 1000)],
                        s_out.at[pl.ds(c * N + s * 1000, 1000)])


def _sc_gat(hh, p, q, pmax, src3, dst3):
    return pl.kernel(
        _gat_body,
        out_type=(jax.ShapeDtypeStruct((NC, N, D), jnp.float32),
                  jax.ShapeDtypeStruct((NC * N,), jnp.float32)),
        mesh=_mesh,
        compiler_params=_sc_params,
        scratch_types=[
            pltpu.VMEM((NBLK, B), jnp.int32),
            pltpu.VMEM((NBLK, B), jnp.int32),
            pltpu.VMEM((KB, B, D), jnp.bfloat16),
            pltpu.VMEM((KB, B, D), jnp.float32),
            pltpu.VMEM((B,), jnp.float32),
            pltpu.VMEM((N,), jnp.float32),
            pltpu.VMEM((N,), jnp.float32),
            pltpu.VMEM((16,), jnp.float32),
            pltpu.VMEM((1008,), jnp.float32),
            pltpu.VMEM((125, D), jnp.float32),
            pltpu.VMEM_SHARED((N, D), jnp.float32),
            pltpu.VMEM_SHARED((N,), jnp.float32),
            pltpu.SemaphoreType.DMA((KB,)),
            pltpu.SemaphoreType.DMA((KB,)),
        ],
    )(hh, p, q, pmax, src3, dst3)


def _tc1_body(x_ref, w1_ref, w1w_ref, da_ref, db_ref,
              g1_ref, g1w_ref, dinv_ref):
    deg = da_ref[...] + db_ref[...] + 1.0
    dv = lax.rsqrt(deg)
    xb = x_ref[...]
    g1_ref[...] = jnp.dot(xb, w1_ref[...],
                          preferred_element_type=jnp.float32) * dv
    g1w_ref[...] = (jnp.dot(xb, w1w_ref[...],
                            preferred_element_type=jnp.float32)
                    * dv).astype(jnp.bfloat16)
    dinv_ref[...] = dv


def _tc1(x, W1, W1w, dA, dB):
    return pl.pallas_call(
        _tc1_body,
        grid=(NG,),
        in_specs=[
            pl.BlockSpec((R, DIN), lambda i: (i, 0)),
            pl.BlockSpec((DIN, D), lambda i: (0, 0)),
            pl.BlockSpec((DIN, D), lambda i: (0, 0)),
            pl.BlockSpec((R, 1), lambda i: (i, 0)),
            pl.BlockSpec((R, 1), lambda i: (i, 0)),
        ],
        out_specs=[
            pl.BlockSpec((R, D), lambda i: (i, 0)),
            pl.BlockSpec((R, D), lambda i: (i, 0)),
            pl.BlockSpec((R, 1), lambda i: (i, 0)),
        ],
        out_shape=[
            jax.ShapeDtypeStruct((N, D), jnp.float32),
            jax.ShapeDtypeStruct((N, D), jnp.bfloat16),
            jax.ShapeDtypeStruct((N, 1), jnp.float32),
        ],
    )(x, W1, W1w, dA, dB)


def _tc2_body(accp_ref, g1_ref, dinv_ref, b1_ref, w2_ref, w2w_ref,
              as_ref, ad_ref,
              hh_ref, hhw_ref, p_ref, q_ref, pmax_ref):
    i = pl.program_id(0)
    acc = accp_ref[0] + accp_ref[1] + g1_ref[...]
    h1 = jnp.maximum(dinv_ref[...] * acc + b1_ref[...], 0.0)
    hh = jnp.dot(h1, w2_ref[...], preferred_element_type=jnp.float32)
    hh_ref[...] = hh
    hhw_ref[...] = jnp.dot(h1, w2w_ref[...],
                           preferred_element_type=jnp.float32
                           ).astype(jnp.bfloat16)
    p = jnp.dot(hh, as_ref[...], preferred_element_type=jnp.float32)
    q = jnp.dot(hh, ad_ref[...], preferred_element_type=jnp.float32)
    p_ref[...] = p
    q_ref[...] = q
    pb = jnp.max(p, axis=(0, 1), keepdims=True)

    @pl.when(i == 0)
    def _():
        pmax_ref[...] = pb

    @pl.when(i > 0)
    def _():
        pmax_ref[...] = jnp.maximum(pmax_ref[...], pb)


def _tc2(accP, g1, dinv, b1, W2, W2w, aS, aD):
    return pl.pallas_call(
        _tc2_body,
        grid=(NG,),
        in_specs=[
            pl.BlockSpec((NC, R, D), lambda i: (0, i, 0)),
            pl.BlockSpec((R, D), lambda i: (i, 0)),
            pl.BlockSpec((R, 1), lambda i: (i, 0)),
            pl.BlockSpec((1, D), lambda i: (0, 0)),
            pl.BlockSpec((D, D), lambda i: (0, 0)),
            pl.BlockSpec((D, D), lambda i: (0, 0)),
            pl.BlockSpec((D, 1), lambda i: (0, 0)),
            pl.BlockSpec((D, 1), lambda i: (0, 0)),
        ],
        out_specs=[
            pl.BlockSpec((R, D), lambda i: (i, 0)),
            pl.BlockSpec((R, D), lambda i: (i, 0)),
            pl.BlockSpec((R, 1), lambda i: (i, 0)),
            pl.BlockSpec((R, 1), lambda i: (i, 0)),
            pl.BlockSpec((1, 1), lambda i: (0, 0)),
        ],
        out_shape=[
            jax.ShapeDtypeStruct((N, D), jnp.float32),
            jax.ShapeDtypeStruct((N, D), jnp.bfloat16),
            jax.ShapeDtypeStruct((N, 1), jnp.float32),
            jax.ShapeDtypeStruct((N, 1), jnp.float32),
            jax.ShapeDtypeStruct((1, 1), jnp.float32),
        ],
    )(accP, g1, dinv, b1, W2, W2w, aS, aD)


def _tc3_body(accp_ref, sp_ref, hh_ref, p_ref, q_ref, pmax_ref, dinv_ref,
              b2_ref, w3_ref, w3w_ref, g3_ref, g3w_ref):
    p = p_ref[...]
    q = q_ref[...]
    z = p + q
    e_self = jnp.maximum(z, 0.2 * z)
    zu = pmax_ref[0, 0] + q
    u = jnp.maximum(zu, 0.2 * zu)
    w_self = jnp.exp(e_self - u)
    den = sp_ref[0] + sp_ref[1] + w_self + 1e-16
    num = accp_ref[0] + accp_ref[1] + w_self * hh_ref[...]
    h2 = jnp.maximum(num / den + b2_ref[...], 0.0)
    dv = dinv_ref[...]
    g3_ref[...] = jnp.dot(h2, w3_ref[...],
                          preferred_element_type=jnp.float32) * dv
    g3w_ref[...] = (jnp.dot(h2, w3w_ref[...],
                            preferred_element_type=jnp.float32)
                    * dv).astype(jnp.bfloat16)


def _tc3(accP, sP, hh, p, q, pmax, dinv, b2, W3, W3w):
    return pl.pallas_call(
        _tc3_body,
        grid=(NG,),
        in_specs=[
            pl.BlockSpec((NC, R, D), lambda i: (0, i, 0)),
            pl.BlockSpec((NC, R, 1), lambda i: (0, i, 0)),
            pl.BlockSpec((R, D), lambda i: (i, 0)),
            pl.BlockSpec((R, 1), lambda i: (i, 0)),
            pl.BlockSpec((R, 1), lambda i: (i, 0)),
            pl.BlockSpec((1, 1), lambda i: (0, 0)),
            pl.BlockSpec((R, 1), lambda i: (i, 0)),
            pl.BlockSpec((1, D), lambda i: (0, 0)),
            pl.BlockSpec((D, D), lambda i: (0, 0)),
            pl.BlockSpec((D, D), lambda i: (0, 0)),
        ],
        out_specs=[
            pl.BlockSpec((R, D), lambda i: (i, 0)),
            pl.BlockSpec((R, D), lambda i: (i, 0)),
        ],
        out_shape=[
            jax.ShapeDtypeStruct((N, D), jnp.float32),
            jax.ShapeDtypeStruct((N, D), jnp.bfloat16),
        ],
    )(accP, sP, hh, p, q, pmax, dinv, b2, W3, W3w)


def _tc4_body(accp_ref, g3_ref, dinv_ref, b3_ref, wo_ref, bo_ref,
              wb1_ref, bb1_ref, wb2_ref, bb2_ref,
              opt_ref, bt_ref, ge_ref):
    i = pl.program_id(0)
    acc = accp_ref[0] + accp_ref[1] + g3_ref[...]
    h3 = jnp.maximum(dinv_ref[...] * acc + b3_ref[...], 0.0)
    opt_ref[...] = jnp.dot(h3, wo_ref[...],
                           preferred_element_type=jnp.float32) + bo_ref[...]
    t = jnp.maximum(jnp.dot(h3, wb1_ref[...],
                            preferred_element_type=jnp.float32) + bb1_ref[...],
                    0.0)
    bt_ref[...] = jax.nn.sigmoid(
        jnp.dot(t, wb2_ref[...], preferred_element_type=jnp.float32)
        + bb2_ref[...])
    tot = jnp.sum(h3, axis=0, keepdims=True)

    @pl.when(i == 0)
    def _():
        ge_ref[...] = tot

    @pl.when(i > 0)
    def _():
        ge_ref[...] = ge_ref[...] + tot

    @pl.when(i == NG - 1)
    def _():
        ge_ref[...] = ge_ref[...] * (1.0 / N)


def _tc4(accP, g3, dinv, b3, Wo, bo, Wb1, bb1, Wb2, bb2):
    return pl.pallas_call(
        _tc4_body,
        grid=(NG,),
        in_specs=[
            pl.BlockSpec((NC, R, D), lambda i: (0, i, 0)),
            pl.BlockSpec((R, D), lambda i: (i, 0)),
            pl.BlockSpec((R, 1), lambda i: (i, 0)),
            pl.BlockSpec((1, D), lambda i: (0, 0)),
            pl.BlockSpec((D, NCLS), lambda i: (0, 0)),
            pl.BlockSpec((1, NCLS), lambda i: (0, 0)),
            pl.BlockSpec((D, 32), lambda i: (0, 0)),
            pl.BlockSpec((1, 32), lambda i: (0, 0)),
            pl.BlockSpec((32, 1), lambda i: (0, 0)),
            pl.BlockSpec((1, 1), lambda i: (0, 0)),
        ],
        out_specs=[
            pl.BlockSpec((R, NCLS), lambda i: (i, 0)),
            pl.BlockSpec((R, 1), lambda i: (i, 0)),
            pl.BlockSpec((1, D), lambda i: (0, 0)),
        ],
        out_shape=[
            jax.ShapeDtypeStruct((N, NCLS), jnp.float32),
            jax.ShapeDtypeStruct((N, 1), jnp.float32),
            jax.ShapeDtypeStruct((1, D), jnp.float32),
        ],
    )(accP, g3, dinv, b3, Wo, bo, Wb1, bb1, Wb2, bb2)


def kernel(x, edge_index, W1, b1, W2, a_src, a_dst, b2, W3, b3, Wo, bo,
           Wb1, bb1, Wb2, bb2):
    src3 = edge_index[0].reshape(NW, NBLK, B)
    dst3 = edge_index[1].reshape(NW, NBLK, B)

    degP = _sc_deg(dst3).reshape(NC, N)
    dA = degP[0].reshape(N, 1)
    dB = degP[1].reshape(N, 1)

    inv_tau = jnp.asarray(_INV_TAU)
    g1, g1w, dinv = _tc1(x, W1, W1[:, inv_tau], dA, dB)
    acc1 = _sc_gcn(g1w, src3, dst3)
    hh, hhw, p, q, pmax = _tc2(acc1, g1, dinv, b1.reshape(1, D),
                               W2, W2[:, inv_tau],
                               a_src.reshape(D, 1), a_dst.reshape(D, 1))

    pmax16 = jnp.broadcast_to(pmax.reshape(1), (16,))
    acc2, s2 = _sc_gat(hhw, p.reshape(N), q.reshape(N), pmax16,
                       src3, dst3)
    g3, g3w = _tc3(acc2, s2.reshape(NC, N, 1), hh, p, q, pmax, dinv,
                   b2.reshape(1, D), W3, W3[:, inv_tau])

    acc3 = _sc_gcn(g3w, src3, dst3)
    opt, bt, ge = _tc4(acc3, g3, dinv, b3.reshape(1, D), Wo,
                       bo.reshape(1, NCLS), Wb1, bb1.reshape(1, 32),
                       Wb2, bb2.reshape(1, 1))
    return opt, bt, ge.reshape(D)


# speedup vs baseline: 1.0795x; 1.0389x over previous
import functools

import jax
import jax.numpy as jnp
import numpy as np
from jax import lax
from jax.experimental import pallas as pl
from jax.experimental.pallas import tpu as pltpu
from jax.experimental.pallas import tpu_sc as plsc

N = 10000
E = 320000
DIN = 128
D = 64
NCLS = 10

NC = 2
NS = 16
NW = NC * NS
EPT = E // NW
B = 80
NBLK = EPT // B

KB = 4
KG = 3
DP = D + 16

R = 2000
NG = N // R


_TAU = np.empty(D, np.int32)
for _c in range(2):
    for _k in range(16):
        _TAU[_c * 32 + _k] = _c * 32 + 2 * _k
        _TAU[_c * 32 + 16 + _k] = _c * 32 + 2 * _k + 1
_INV_TAU = np.argsort(_TAU)

_mesh = plsc.VectorSubcoreMesh(
    core_axis_name="c", subcore_axis_name="s", num_cores=NC, num_subcores=NS)
_sc_params = pltpu.CompilerParams(use_tc_tiling_on_sc=False,
                                  needs_layout_passes=False)


def _zero_vec(buf, nv):
    def zb(t, carry):
        buf[pl.ds(t * 16, 16)] = jnp.zeros((16,), jnp.float32)
        return carry
    lax.fori_loop(0, nv, zb, 0)


def _deg_body(dst3_hbm, out_hbm, dst_idx, ones_v, buf1, deg_sh, sem):
    c = lax.axis_index("c")
    s = lax.axis_index("s")
    chunk = c * NS + s
    pltpu.sync_copy(dst3_hbm.at[chunk], dst_idx)
    for k in range(B // 16):
        ones_v[pl.ds(k * 16, 16)] = jnp.ones((16,), jnp.float32)
    _zero_vec(buf1, 63)

    @pl.when(s < 10)
    def _():
        pltpu.sync_copy(buf1.at[pl.ds(0, 1000)],
                        deg_sh.at[pl.ds(s * 1000, 1000)])

    plsc.subcore_barrier()

    def blk(j, carry):
        pltpu.sync_copy(ones_v, deg_sh.at[dst_idx.at[j]], add=True)
        return carry

    lax.fori_loop(0, NBLK, blk, 0)
    plsc.subcore_barrier()

    @pl.when(s < 10)
    def _():
        pltpu.sync_copy(deg_sh.at[pl.ds(s * 1000, 1000)],
                        buf1.at[pl.ds(0, 1000)])
        pltpu.sync_copy(buf1.at[pl.ds(0, 1000)],
                        out_hbm.at[pl.ds(c * N + s * 1000, 1000)])


def _sc_deg(dst3):
    return pl.kernel(
        _deg_body,
        out_type=jax.ShapeDtypeStruct((NC * N,), jnp.float32),
        mesh=_mesh,
        compiler_params=_sc_params,
        scratch_types=[
            pltpu.VMEM((NBLK, B), jnp.int32),
            pltpu.VMEM((B,), jnp.float32),
            pltpu.VMEM((1008,), jnp.float32),
            pltpu.VMEM_SHARED((N,), jnp.float32),
            pltpu.SemaphoreType.DMA,
        ],
    )(dst3)


def _zero_rows(buf, nrows, width):
    def zb(r, carry):
        for t in range(width // 16):
            buf[r, pl.ds(t * 16, 16)] = jnp.zeros((16,), jnp.float32)
        return carry
    lax.fori_loop(0, nrows, zb, 0)


def _acc_init(acc_sh, bounce, s, width):
    _zero_rows(bounce, 125, width)

    def zc(ch, carry):
        pltpu.sync_copy(bounce, acc_sh.at[pl.ds(s * 625 + ch * 125, 125)])
        return carry

    lax.fori_loop(0, 5, zc, 0)


def _acc_out(acc_sh, bounce, out_hbm, c, s):
    def oc(ch, carry):
        r0 = s * 625 + ch * 125
        pltpu.sync_copy(acc_sh.at[pl.ds(r0, 125)], bounce)
        pltpu.sync_copy(bounce, out_hbm.at[c, pl.ds(r0, 125)])
        return carry

    lax.fori_loop(0, 5, oc, 0)


def _unpack_row(rows_bf, rowsf, b, i, bw=None):
    for c2 in range(2):
        v = plsc.bitcast(rows_bf[b, i, pl.ds(c2 * 32, 32)], jnp.int32)
        lo = plsc.bitcast(jnp.left_shift(v, 16), jnp.float32)
        hi = plsc.bitcast(jnp.bitwise_and(v, jnp.int32(-65536)), jnp.float32)
        if bw is not None:
            lo = lo * bw
            hi = hi * bw
        rowsf[b, i, pl.ds(c2 * 32, 16)] = lo
        rowsf[b, i, pl.ds(c2 * 32 + 16, 16)] = hi


def _gcn_body(g_hbm, src3_hbm, dst3_hbm, out_hbm,
              src_idx, dst_idx, rows_bf, rowsf, big_buf, acc_sh,
              gsems, ssems):
    c = lax.axis_index("c")
    s = lax.axis_index("s")
    chunk = c * NS + s
    pltpu.sync_copy(src3_hbm.at[chunk], src_idx)
    pltpu.sync_copy(dst3_hbm.at[chunk], dst_idx)
    _acc_init(acc_sh, big_buf, s, D)
    plsc.subcore_barrier()
    pltpu.async_copy(g_hbm.at[src_idx.at[0]], rows_bf.at[0], gsems.at[0])
    pltpu.async_copy(g_hbm.at[src_idx.at[1]], rows_bf.at[1], gsems.at[1])

    def blk(j, carry):
        b = lax.rem(j, KB)
        pltpu.make_async_copy(g_hbm.at[src_idx.at[j]], rows_bf.at[b],
                              gsems.at[b]).wait()

        @plsc.parallel_loop(0, B, unroll=4)
        def _(i):
            _unpack_row(rows_bf, rowsf, b, i)

        pltpu.async_copy(rowsf.at[b], acc_sh.at[dst_idx.at[j]], ssems.at[b],
                         add=True)

        @pl.when(j + 2 < NBLK)
        def _():
            b2 = lax.rem(j + 2, KB)

            @pl.when(j >= 2)
            def _():
                pltpu.make_async_copy(rowsf.at[b2],
                                      acc_sh.at[dst_idx.at[j - 2]],
                                      ssems.at[b2]).wait()

            pltpu.async_copy(g_hbm.at[src_idx.at[j + 2]], rows_bf.at[b2],
                             gsems.at[b2])

        return carry

    lax.fori_loop(0, NBLK, blk, 0)

    def drain(t, carry):
        jj = NBLK - KB + t
        bb = lax.rem(jj, KB)
        pltpu.make_async_copy(rowsf.at[bb], acc_sh.at[dst_idx.at[jj]],
                              ssems.at[bb]).wait()
        return carry

    lax.fori_loop(0, KB, drain, 0)
    plsc.subcore_barrier()
    _acc_out(acc_sh, big_buf, out_hbm, c, s)


def _sc_gcn(g, src3, dst3):
    return pl.kernel(
        _gcn_body,
        out_type=jax.ShapeDtypeStruct((NC, N, D), jnp.float32),
        mesh=_mesh,
        compiler_params=_sc_params,
        scratch_types=[
            pltpu.VMEM((NBLK, B), jnp.int32),
            pltpu.VMEM((NBLK, B), jnp.int32),
            pltpu.VMEM((KB, B, D), jnp.bfloat16),
            pltpu.VMEM((KB, B, D), jnp.float32),
            pltpu.VMEM((125, D), jnp.float32),
            pltpu.VMEM_SHARED((N, D), jnp.float32),
            pltpu.SemaphoreType.DMA((KB,)),
            pltpu.SemaphoreType.DMA((KB,)),
        ],
    )(g, src3, dst3)


def _gat_body(hh_hbm, p_hbm, q_hbm, pmax_hbm, src3_hbm, dst3_hbm,
              acc_out,
              src_idx, dst_idx, rows_bf, rowsf, w_buf, p_v, q_v, pm_v,
              big_buf, acc_sh, gsems, ssems):
    c = lax.axis_index("c")
    s = lax.axis_index("s")
    chunk = c * NS + s
    pltpu.sync_copy(src3_hbm.at[chunk], src_idx)
    pltpu.sync_copy(dst3_hbm.at[chunk], dst_idx)
    pltpu.sync_copy(p_hbm, p_v)
    pltpu.sync_copy(q_hbm, q_v)
    pltpu.sync_copy(pmax_hbm, pm_v)
    _acc_init(acc_sh, big_buf, s, DP)
    plsc.subcore_barrier()
    pmv = pm_v[...]
    e0 = jnp.where(lax.iota(jnp.int32, 16) == 0, 1.0, 0.0)
    pltpu.async_copy(hh_hbm.at[src_idx.at[0]], rows_bf.at[0], gsems.at[0])
    pltpu.async_copy(hh_hbm.at[src_idx.at[1]], rows_bf.at[1], gsems.at[1])

    def blk(j, carry):
        b = lax.rem(j, KG)
        for k in range(B // 16):
            si = src_idx[j, pl.ds(k * 16, 16)]
            di = dst_idx[j, pl.ds(k * 16, 16)]
            pv = plsc.load_gather(p_v, [si])
            qv = plsc.load_gather(q_v, [di])
            z = pv + qv
            e = jnp.maximum(z, 0.2 * z)
            zu = pmv + qv
            u = jnp.maximum(zu, 0.2 * zu)
            w_buf[pl.ds(k * 16, 16)] = jnp.exp(e - u)

        pltpu.make_async_copy(hh_hbm.at[src_idx.at[j]], rows_bf.at[b],
                              gsems.at[b]).wait()

        @plsc.parallel_loop(0, B, unroll=4)
        def _(i):
            bw = plsc.load_gather(w_buf, [jnp.full((16,), i, jnp.int32)])
            _unpack_row(rows_bf, rowsf, b, i, bw)
            rowsf[b, i, pl.ds(D, 16)] = bw * e0

        pltpu.async_copy(rowsf.at[b], acc_sh.at[dst_idx.at[j]], ssems.at[b],
                         add=True)

        @pl.when(j + 2 < NBLK)
        def _():
            b2 = lax.rem(j + 2, KG)

            @pl.when(j >= 1)
            def _():
                pltpu.make_async_copy(rowsf.at[b2],
                                      acc_sh.at[dst_idx.at[j - 1]],
                                      ssems.at[b2]).wait()

            pltpu.async_copy(hh_hbm.at[src_idx.at[j + 2]], rows_bf.at[b2],
                             gsems.at[b2])

        return carry

    lax.fori_loop(0, NBLK, blk, 0)

    def drain(t, carry):
        jj = NBLK - 3 + t
        bb = lax.rem(jj, KG)
        pltpu.make_async_copy(rowsf.at[bb], acc_sh.at[dst_idx.at[jj]],
                              ssems.at[bb]).wait()
        return carry

    lax.fori_loop(0, 3, drain, 0)
    plsc.subcore_barrier()
    _acc_out(acc_sh, big_buf, acc_out, c, s)


def _sc_gat(hh, p, q, pmax, src3, dst3):
    return pl.kernel(
        _gat_body,
        out_type=jax.ShapeDtypeStruct((NC, N, DP), jnp.float32),
        mesh=_mesh,
        compiler_params=_sc_params,
        scratch_types=[
            pltpu.VMEM((NBLK, B), jnp.int32),
            pltpu.VMEM((NBLK, B), jnp.int32),
            pltpu.VMEM((KG, B, D), jnp.bfloat16),
            pltpu.VMEM((KG, B, DP), jnp.float32),
            pltpu.VMEM((B,), jnp.float32),
            pltpu.VMEM((N,), jnp.float32),
            pltpu.VMEM((N,), jnp.float32),
            pltpu.VMEM((16,), jnp.float32),
            pltpu.VMEM((125, DP), jnp.float32),
            pltpu.VMEM_SHARED((N, DP), jnp.float32),
            pltpu.SemaphoreType.DMA((KG,)),
            pltpu.SemaphoreType.DMA((KG,)),
        ],
    )(hh, p, q, pmax, src3, dst3)


def _tc1a_body(x_ref, w1_ref, w1w_ref, xw_ref, xww_ref):
    xb = x_ref[...]
    xw_ref[...] = jnp.dot(xb, w1_ref[...],
                          preferred_element_type=jnp.float32)
    xww_ref[...] = jnp.dot(xb, w1w_ref[...],
                           preferred_element_type=jnp.float32)


def _tc1a(x, W1, W1w):
    return pl.pallas_call(
        _tc1a_body,
        grid=(NG,),
        in_specs=[
            pl.BlockSpec((R, DIN), lambda i: (i, 0)),
            pl.BlockSpec((DIN, D), lambda i: (0, 0)),
            pl.BlockSpec((DIN, D), lambda i: (0, 0)),
        ],
        out_specs=[
            pl.BlockSpec((R, D), lambda i: (i, 0)),
            pl.BlockSpec((R, D), lambda i: (i, 0)),
        ],
        out_shape=[
            jax.ShapeDtypeStruct((N, D), jnp.float32),
            jax.ShapeDtypeStruct((N, D), jnp.float32),
        ],
    )(x, W1, W1w)


def _tc1b_body(xw_ref, xww_ref, da_ref, db_ref, g1_ref, g1w_ref, dinv_ref):
    deg = da_ref[...] + db_ref[...] + 1.0
    dv = lax.rsqrt(deg)
    g1_ref[...] = xw_ref[...] * dv
    g1w_ref[...] = (xww_ref[...] * dv).astype(jnp.bfloat16)
    dinv_ref[...] = dv


def _tc1b(xw, xww, dA, dB):
    return pl.pallas_call(
        _tc1b_body,
        grid=(NG,),
        in_specs=[
            pl.BlockSpec((R, D), lambda i: (i, 0)),
            pl.BlockSpec((R, D), lambda i: (i, 0)),
            pl.BlockSpec((R, 1), lambda i: (i, 0)),
            pl.BlockSpec((R, 1), lambda i: (i, 0)),
        ],
        out_specs=[
            pl.BlockSpec((R, D), lambda i: (i, 0)),
            pl.BlockSpec((R, D), lambda i: (i, 0)),
            pl.BlockSpec((R, 1), lambda i: (i, 0)),
        ],
        out_shape=[
            jax.ShapeDtypeStruct((N, D), jnp.float32),
            jax.ShapeDtypeStruct((N, D), jnp.bfloat16),
            jax.ShapeDtypeStruct((N, 1), jnp.float32),
        ],
    )(xw, xww, dA, dB)


def _tc2_body(accp_ref, g1_ref, dinv_ref, b1_ref, w2_ref, w2w_ref,
              as_ref, ad_ref,
              hh_ref, hhw_ref, p_ref, q_ref, pmax_ref):
    i = pl.program_id(0)
    acc = accp_ref[0] + accp_ref[1] + g1_ref[...]
    h1 = jnp.maximum(dinv_ref[...] * acc + b1_ref[...], 0.0)
    hh = jnp.dot(h1, w2_ref[...], preferred_element_type=jnp.float32)
    hh_ref[...] = hh
    hhw_ref[...] = jnp.dot(h1, w2w_ref[...],
                           preferred_element_type=jnp.float32
                           ).astype(jnp.bfloat16)
    p = jnp.dot(hh, as_ref[...], preferred_element_type=jnp.float32)
    q = jnp.dot(hh, ad_ref[...], preferred_element_type=jnp.float32)
    p_ref[...] = p
    q_ref[...] = q
    pb = jnp.max(p, axis=(0, 1), keepdims=True)

    @pl.when(i == 0)
    def _():
        pmax_ref[...] = pb

    @pl.when(i > 0)
    def _():
        pmax_ref[...] = jnp.maximum(pmax_ref[...], pb)


def _tc2(accP, g1, dinv, b1, W2, W2w, aS, aD):
    return pl.pallas_call(
        _tc2_body,
        grid=(NG,),
        in_specs=[
            pl.BlockSpec((NC, R, D), lambda i: (0, i, 0)),
            pl.BlockSpec((R, D), lambda i: (i, 0)),
            pl.BlockSpec((R, 1), lambda i: (i, 0)),
            pl.BlockSpec((1, D), lambda i: (0, 0)),
            pl.BlockSpec((D, D), lambda i: (0, 0)),
            pl.BlockSpec((D, D), lambda i: (0, 0)),
            pl.BlockSpec((D, 1), lambda i: (0, 0)),
            pl.BlockSpec((D, 1), lambda i: (0, 0)),
        ],
        out_specs=[
            pl.BlockSpec((R, D), lambda i: (i, 0)),
            pl.BlockSpec((R, D), lambda i: (i, 0)),
            pl.BlockSpec((R, 1), lambda i: (i, 0)),
            pl.BlockSpec((R, 1), lambda i: (i, 0)),
            pl.BlockSpec((1, 1), lambda i: (0, 0)),
        ],
        out_shape=[
            jax.ShapeDtypeStruct((N, D), jnp.float32),
            jax.ShapeDtypeStruct((N, D), jnp.bfloat16),
            jax.ShapeDtypeStruct((N, 1), jnp.float32),
            jax.ShapeDtypeStruct((N, 1), jnp.float32),
            jax.ShapeDtypeStruct((1, 1), jnp.float32),
        ],
    )(accP, g1, dinv, b1, W2, W2w, aS, aD)


def _tc3_body(accp_ref, hh_ref, p_ref, q_ref, pmax_ref, dinv_ref,
              b2_ref, w3_ref, w3w_ref, g3_ref, g3w_ref):
    p = p_ref[...]
    q = q_ref[...]
    z = p + q
    e_self = jnp.maximum(z, 0.2 * z)
    zu = pmax_ref[0, 0] + q
    u = jnp.maximum(zu, 0.2 * zu)
    w_self = jnp.exp(e_self - u)
    a0 = accp_ref[0]
    a1 = accp_ref[1]
    den = a0[:, D:D + 1] + a1[:, D:D + 1] + w_self + 1e-16
    num = a0[:, :D] + a1[:, :D] + w_self * hh_ref[...]
    h2 = jnp.maximum(num / den + b2_ref[...], 0.0)
    dv = dinv_ref[...]
    g3_ref[...] = jnp.dot(h2, w3_ref[...],
                          preferred_element_type=jnp.float32) * dv
    g3w_ref[...] = (jnp.dot(h2, w3w_ref[...],
                            preferred_element_type=jnp.float32)
                    * dv).astype(jnp.bfloat16)


def _tc3(accP, hh, p, q, pmax, dinv, b2, W3, W3w):
    return pl.pallas_call(
        _tc3_body,
        grid=(NG,),
        in_specs=[
            pl.BlockSpec((NC, R, DP), lambda i: (0, i, 0)),
            pl.BlockSpec((R, D), lambda i: (i, 0)),
            pl.BlockSpec((R, 1), lambda i: (i, 0)),
            pl.BlockSpec((R, 1), lambda i: (i, 0)),
            pl.BlockSpec((1, 1), lambda i: (0, 0)),
            pl.BlockSpec((R, 1), lambda i: (i, 0)),
            pl.BlockSpec((1, D), lambda i: (0, 0)),
            pl.BlockSpec((D, D), lambda i: (0, 0)),
            pl.BlockSpec((D, D), lambda i: (0, 0)),
        ],
        out_specs=[
            pl.BlockSpec((R, D), lambda i: (i, 0)),
            pl.BlockSpec((R, D), lambda i: (i, 0)),
        ],
        out_shape=[
            jax.ShapeDtypeStruct((N, D), jnp.float32),
            jax.ShapeDtypeStruct((N, D), jnp.bfloat16),
        ],
    )(accP, hh, p, q, pmax, dinv, b2, W3, W3w)


def _tc4_body(accp_ref, g3_ref, dinv_ref, b3_ref, wo_ref, bo_ref,
              wb1_ref, bb1_ref, wb2_ref, bb2_ref,
              opt_ref, bt_ref, ge_ref):
    i = pl.program_id(0)
    acc = accp_ref[0] + accp_ref[1] + g3_ref[...]
    h3 = jnp.maximum(dinv_ref[...] * acc + b3_ref[...], 0.0)
    opt_ref[...] = jnp.dot(h3, wo_ref[...],
                           preferred_element_type=jnp.float32) + bo_ref[...]
    t = jnp.maximum(jnp.dot(h3, wb1_ref[...],
                            preferred_element_type=jnp.float32) + bb1_ref[...],
                    0.0)
    bt_ref[...] = jax.nn.sigmoid(
        jnp.dot(t, wb2_ref[...], preferred_element_type=jnp.float32)
        + bb2_ref[...])
    tot = jnp.sum(h3, axis=0, keepdims=True)

    @pl.when(i == 0)
    def _():
        ge_ref[...] = tot

    @pl.when(i > 0)
    def _():
        ge_ref[...] = ge_ref[...] + tot

    @pl.when(i == NG - 1)
    def _():
        ge_ref[...] = ge_ref[...] * (1.0 / N)


def _tc4(accP, g3, dinv, b3, Wo, bo, Wb1, bb1, Wb2, bb2):
    return pl.pallas_call(
        _tc4_body,
        grid=(NG,),
        in_specs=[
            pl.BlockSpec((NC, R, D), lambda i: (0, i, 0)),
            pl.BlockSpec((R, D), lambda i: (i, 0)),
            pl.BlockSpec((R, 1), lambda i: (i, 0)),
            pl.BlockSpec((1, D), lambda i: (0, 0)),
            pl.BlockSpec((D, NCLS), lambda i: (0, 0)),
            pl.BlockSpec((1, NCLS), lambda i: (0, 0)),
            pl.BlockSpec((D, 32), lambda i: (0, 0)),
            pl.BlockSpec((1, 32), lambda i: (0, 0)),
            pl.BlockSpec((32, 1), lambda i: (0, 0)),
            pl.BlockSpec((1, 1), lambda i: (0, 0)),
        ],
        out_specs=[
            pl.BlockSpec((R, NCLS), lambda i: (i, 0)),
            pl.BlockSpec((R, 1), lambda i: (i, 0)),
            pl.BlockSpec((1, D), lambda i: (0, 0)),
        ],
        out_shape=[
            jax.ShapeDtypeStruct((N, NCLS), jnp.float32),
            jax.ShapeDtypeStruct((N, 1), jnp.float32),
            jax.ShapeDtypeStruct((1, D), jnp.float32),
        ],
    )(accP, g3, dinv, b3, Wo, bo, Wb1, bb1, Wb2, bb2)


def kernel(x, edge_index, W1, b1, W2, a_src, a_dst, b2, W3, b3, Wo, bo,
           Wb1, bb1, Wb2, bb2):
    src3 = edge_index[0].reshape(NW, NBLK, B)
    dst3 = edge_index[1].reshape(NW, NBLK, B)

    degP = _sc_deg(dst3).reshape(NC, N)
    dA = degP[0].reshape(N, 1)
    dB = degP[1].reshape(N, 1)

    inv_tau = jnp.asarray(_INV_TAU)
    xw, xww = _tc1a(x, W1, W1[:, inv_tau])
    g1, g1w, dinv = _tc1b(xw, xww, dA, dB)
    acc1 = _sc_gcn(g1w, src3, dst3)
    hh, hhw, p, q, pmax = _tc2(acc1, g1, dinv, b1.reshape(1, D),
                               W2, W2[:, inv_tau],
                               a_src.reshape(D, 1), a_dst.reshape(D, 1))

    pmax16 = jnp.broadcast_to(pmax.reshape(1), (16,))
    acc2 = _sc_gat(hhw, p.reshape(N), q.reshape(N), pmax16, src3, dst3)
    g3, g3w = _tc3(acc2, hh, p, q, pmax, dinv,
                   b2.reshape(1, D), W3, W3[:, inv_tau])

    acc3 = _sc_gcn(g3w, src3, dst3)
    opt, bt, ge = _tc4(acc3, g3, dinv, b3.reshape(1, D), Wo,
                       bo.reshape(1, NCLS), Wb1, bb1.reshape(1, 32),
                       Wb2, bb2.reshape(1, 1))
    return opt, bt, ge.reshape(D)


# speedup vs baseline: 1.0849x; 1.0050x over previous
import functools

import jax
import jax.numpy as jnp
import numpy as np
from jax import lax
from jax.experimental import pallas as pl
from jax.experimental.pallas import tpu as pltpu
from jax.experimental.pallas import tpu_sc as plsc

N = 10000
E = 320000
DIN = 128
D = 64
NCLS = 10

NC = 2
NS = 16
NW = NC * NS
EPT = E // NW
B = 80
NBLK = EPT // B

KB = 4
KG = 3
DP = D + 16

R = 2000
NG = N // R


_TAU = np.empty(D, np.int32)
for _c in range(2):
    for _k in range(16):
        _TAU[_c * 32 + _k] = _c * 32 + 2 * _k
        _TAU[_c * 32 + 16 + _k] = _c * 32 + 2 * _k + 1
_INV_TAU = np.argsort(_TAU)

_mesh = plsc.VectorSubcoreMesh(
    core_axis_name="c", subcore_axis_name="s", num_cores=NC, num_subcores=NS)
_sc_params = pltpu.CompilerParams(use_tc_tiling_on_sc=False,
                                  needs_layout_passes=False)


def _zero_vec(buf, nv):
    def zb(t, carry):
        buf[pl.ds(t * 16, 16)] = jnp.zeros((16,), jnp.float32)
        return carry
    lax.fori_loop(0, nv, zb, 0)


def _deg_body(dst3_hbm, out_hbm, dst_idx, ones_v, buf1, deg_sh, sem):
    c = lax.axis_index("c")
    s = lax.axis_index("s")
    chunk = c * NS + s
    pltpu.sync_copy(dst3_hbm.at[chunk], dst_idx)
    for k in range(B // 16):
        ones_v[pl.ds(k * 16, 16)] = jnp.ones((16,), jnp.float32)
    _zero_vec(buf1, 63)

    @pl.when(s < 10)
    def _():
        pltpu.sync_copy(buf1.at[pl.ds(0, 1000)],
                        deg_sh.at[pl.ds(s * 1000, 1000)])

    plsc.subcore_barrier()

    def blk(j, carry):
        pltpu.sync_copy(ones_v, deg_sh.at[dst_idx.at[j]], add=True)
        return carry

    lax.fori_loop(0, NBLK, blk, 0)
    plsc.subcore_barrier()

    @pl.when(s < 10)
    def _():
        pltpu.sync_copy(deg_sh.at[pl.ds(s * 1000, 1000)],
                        buf1.at[pl.ds(0, 1000)])
        pltpu.sync_copy(buf1.at[pl.ds(0, 1000)],
                        out_hbm.at[pl.ds(c * N + s * 1000, 1000)])


def _sc_deg(dst3):
    return pl.kernel(
        _deg_body,
        out_type=jax.ShapeDtypeStruct((NC * N,), jnp.float32),
        mesh=_mesh,
        compiler_params=_sc_params,
        scratch_types=[
            pltpu.VMEM((NBLK, B), jnp.int32),
            pltpu.VMEM((B,), jnp.float32),
            pltpu.VMEM((1008,), jnp.float32),
            pltpu.VMEM_SHARED((N,), jnp.float32),
            pltpu.SemaphoreType.DMA,
        ],
    )(dst3)


def _zero_rows(buf, nrows, width):
    def zb(r, carry):
        for t in range(width // 16):
            buf[r, pl.ds(t * 16, 16)] = jnp.zeros((16,), jnp.float32)
        return carry
    lax.fori_loop(0, nrows, zb, 0)


def _acc_init(acc_sh, bounce, s, width):
    _zero_rows(bounce, 125, width)

    def zc(ch, carry):
        pltpu.sync_copy(bounce, acc_sh.at[pl.ds(s * 625 + ch * 125, 125)])
        return carry

    lax.fori_loop(0, 5, zc, 0)


def _acc_out(acc_sh, bounce, out_hbm, c, s):
    def oc(ch, carry):
        r0 = s * 625 + ch * 125
        pltpu.sync_copy(acc_sh.at[pl.ds(r0, 125)], bounce)
        pltpu.sync_copy(bounce, out_hbm.at[c, pl.ds(r0, 125)])
        return carry

    lax.fori_loop(0, 5, oc, 0)


def _acc_out_pad128(acc_sh, bounce, bounce_w, out_hbm, c, s):
    def oc(ch, carry):
        r0 = s * 625 + ch * 125
        pltpu.sync_copy(acc_sh.at[pl.ds(r0, 125)], bounce)

        def widen(r, carry2):
            for t in range(D // 16):
                bounce_w[r, pl.ds(t * 16, 16)] = bounce[r, pl.ds(t * 16, 16)]
            return carry2

        lax.fori_loop(0, 125, widen, 0)
        pltpu.sync_copy(bounce_w, out_hbm.at[c, pl.ds(r0, 125)])
        return carry

    lax.fori_loop(0, 5, oc, 0)


def _unpack_row(rows_bf, rowsf, b, i, bw=None):
    for c2 in range(2):
        v = plsc.bitcast(rows_bf[b, i, pl.ds(c2 * 32, 32)], jnp.int32)
        lo = plsc.bitcast(jnp.left_shift(v, 16), jnp.float32)
        hi = plsc.bitcast(jnp.bitwise_and(v, jnp.int32(-65536)), jnp.float32)
        if bw is not None:
            lo = lo * bw
            hi = hi * bw
        rowsf[b, i, pl.ds(c2 * 32, 16)] = lo
        rowsf[b, i, pl.ds(c2 * 32 + 16, 16)] = hi


def _gcn_body(g_hbm, src3_hbm, dst3_hbm, out_hbm,
              src_idx, dst_idx, rows_bf, rowsf, big_buf, bounce_w, acc_sh,
              gsems, ssems):
    c = lax.axis_index("c")
    s = lax.axis_index("s")
    chunk = c * NS + s
    pltpu.sync_copy(src3_hbm.at[chunk], src_idx)
    pltpu.sync_copy(dst3_hbm.at[chunk], dst_idx)
    _acc_init(acc_sh, big_buf, s, D)
    plsc.subcore_barrier()
    pltpu.async_copy(g_hbm.at[src_idx.at[0]], rows_bf.at[0], gsems.at[0])
    pltpu.async_copy(g_hbm.at[src_idx.at[1]], rows_bf.at[1], gsems.at[1])

    def blk(j, carry):
        b = lax.rem(j, KB)
        pltpu.make_async_copy(g_hbm.at[src_idx.at[j]], rows_bf.at[b],
                              gsems.at[b]).wait()

        @plsc.parallel_loop(0, B, unroll=4)
        def _(i):
            _unpack_row(rows_bf, rowsf, b, i)

        pltpu.async_copy(rowsf.at[b], acc_sh.at[dst_idx.at[j]], ssems.at[b],
                         add=True)

        @pl.when(j + 2 < NBLK)
        def _():
            b2 = lax.rem(j + 2, KB)

            @pl.when(j >= 2)
            def _():
                pltpu.make_async_copy(rowsf.at[b2],
                                      acc_sh.at[dst_idx.at[j - 2]],
                                      ssems.at[b2]).wait()

            pltpu.async_copy(g_hbm.at[src_idx.at[j + 2]], rows_bf.at[b2],
                             gsems.at[b2])

        return carry

    lax.fori_loop(0, NBLK, blk, 0)

    def drain(t, carry):
        jj = NBLK - KB + t
        bb = lax.rem(jj, KB)
        pltpu.make_async_copy(rowsf.at[bb], acc_sh.at[dst_idx.at[jj]],
                              ssems.at[bb]).wait()
        return carry

    lax.fori_loop(0, KB, drain, 0)
    plsc.subcore_barrier()
    _acc_out_pad128(acc_sh, big_buf, bounce_w, out_hbm, c, s)


def _sc_gcn(g, src3, dst3):
    return pl.kernel(
        _gcn_body,
        out_type=jax.ShapeDtypeStruct((NC, N, 128), jnp.float32),
        mesh=_mesh,
        compiler_params=_sc_params,
        scratch_types=[
            pltpu.VMEM((NBLK, B), jnp.int32),
            pltpu.VMEM((NBLK, B), jnp.int32),
            pltpu.VMEM((KB, B, D), jnp.bfloat16),
            pltpu.VMEM((KB, B, D), jnp.float32),
            pltpu.VMEM((125, D), jnp.float32),
            pltpu.VMEM((125, 128), jnp.float32),
            pltpu.VMEM_SHARED((N, D), jnp.float32),
            pltpu.SemaphoreType.DMA((KB,)),
            pltpu.SemaphoreType.DMA((KB,)),
        ],
    )(g, src3, dst3)


def _gat_body(hh_hbm, p_hbm, q_hbm, pmax_hbm, src3_hbm, dst3_hbm,
              acc_out,
              src_idx, dst_idx, rows_bf, rowsf, w_buf, p_v, q_v, pm_v,
              big_buf, acc_sh, gsems, ssems):
    c = lax.axis_index("c")
    s = lax.axis_index("s")
    chunk = c * NS + s
    pltpu.sync_copy(src3_hbm.at[chunk], src_idx)
    pltpu.sync_copy(dst3_hbm.at[chunk], dst_idx)
    pltpu.sync_copy(p_hbm, p_v)
    pltpu.sync_copy(q_hbm, q_v)
    pltpu.sync_copy(pmax_hbm, pm_v)
    _acc_init(acc_sh, big_buf, s, DP)
    plsc.subcore_barrier()
    pmv = pm_v[...]
    e0 = jnp.where(lax.iota(jnp.int32, 16) == 0, 1.0, 0.0)
    pltpu.async_copy(hh_hbm.at[src_idx.at[0]], rows_bf.at[0], gsems.at[0])
    pltpu.async_copy(hh_hbm.at[src_idx.at[1]], rows_bf.at[1], gsems.at[1])

    def blk(j, carry):
        b = lax.rem(j, KG)
        for k in range(B // 16):
            si = src_idx[j, pl.ds(k * 16, 16)]
            di = dst_idx[j, pl.ds(k * 16, 16)]
            pv = plsc.load_gather(p_v, [si])
            qv = plsc.load_gather(q_v, [di])
            z = pv + qv
            e = jnp.maximum(z, 0.2 * z)
            zu = pmv + qv
            u = jnp.maximum(zu, 0.2 * zu)
            w_buf[pl.ds(k * 16, 16)] = jnp.exp(e - u)

        pltpu.make_async_copy(hh_hbm.at[src_idx.at[j]], rows_bf.at[b],
                              gsems.at[b]).wait()

        @plsc.parallel_loop(0, B, unroll=4)
        def _(i):
            bw = plsc.load_gather(w_buf, [jnp.full((16,), i, jnp.int32)])
            _unpack_row(rows_bf, rowsf, b, i, bw)
            rowsf[b, i, pl.ds(D, 16)] = bw * e0

        pltpu.async_copy(rowsf.at[b], acc_sh.at[dst_idx.at[j]], ssems.at[b],
                         add=True)

        @pl.when(j + 2 < NBLK)
        def _():
            b2 = lax.rem(j + 2, KG)

            @pl.when(j >= 1)
            def _():
                pltpu.make_async_copy(rowsf.at[b2],
                                      acc_sh.at[dst_idx.at[j - 1]],
                                      ssems.at[b2]).wait()

            pltpu.async_copy(hh_hbm.at[src_idx.at[j + 2]], rows_bf.at[b2],
                             gsems.at[b2])

        return carry

    lax.fori_loop(0, NBLK, blk, 0)

    def drain(t, carry):
        jj = NBLK - 3 + t
        bb = lax.rem(jj, KG)
        pltpu.make_async_copy(rowsf.at[bb], acc_sh.at[dst_idx.at[jj]],
                              ssems.at[bb]).wait()
        return carry

    lax.fori_loop(0, 3, drain, 0)
    plsc.subcore_barrier()
    _acc_out(acc_sh, big_buf, acc_out, c, s)


def _sc_gat(hh, p, q, pmax, src3, dst3):
    return pl.kernel(
        _gat_body,
        out_type=jax.ShapeDtypeStruct((NC, N, DP), jnp.float32),
        mesh=_mesh,
        compiler_params=_sc_params,
        scratch_types=[
            pltpu.VMEM((NBLK, B), jnp.int32),
            pltpu.VMEM((NBLK, B), jnp.int32),
            pltpu.VMEM((KG, B, D), jnp.bfloat16),
            pltpu.VMEM((KG, B, DP), jnp.float32),
            pltpu.VMEM((B,), jnp.float32),
            pltpu.VMEM((N,), jnp.float32),
            pltpu.VMEM((N,), jnp.float32),
            pltpu.VMEM((16,), jnp.float32),
            pltpu.VMEM((125, DP), jnp.float32),
            pltpu.VMEM_SHARED((N, DP), jnp.float32),
            pltpu.SemaphoreType.DMA((KG,)),
            pltpu.SemaphoreType.DMA((KG,)),
        ],
    )(hh, p, q, pmax, src3, dst3)


def _tc1a_body(x_ref, w1_ref, w1w_ref, xw_ref, xww_ref):
    xb = x_ref[...]
    xw_ref[...] = jnp.dot(xb, w1_ref[...],
                          preferred_element_type=jnp.float32)
    xww_ref[...] = jnp.dot(xb, w1w_ref[...],
                           preferred_element_type=jnp.float32)


def _tc1a(x, W1, W1w):
    return pl.pallas_call(
        _tc1a_body,
        grid=(NG,),
        in_specs=[
            pl.BlockSpec((R, DIN), lambda i: (i, 0)),
            pl.BlockSpec((DIN, D), lambda i: (0, 0)),
            pl.BlockSpec((DIN, D), lambda i: (0, 0)),
        ],
        out_specs=[
            pl.BlockSpec((R, D), lambda i: (i, 0)),
            pl.BlockSpec((R, D), lambda i: (i, 0)),
        ],
        out_shape=[
            jax.ShapeDtypeStruct((N, D), jnp.float32),
            jax.ShapeDtypeStruct((N, D), jnp.float32),
        ],
    )(x, W1, W1w)


def _tc1b_body(xww_ref, da_ref, db_ref, g1w_ref, dinv_ref):
    deg = da_ref[...] + db_ref[...] + 1.0
    dv = lax.rsqrt(deg)
    g1w_ref[...] = (xww_ref[...] * dv).astype(jnp.bfloat16)
    dinv_ref[...] = dv


def _tc1b(xww, dA, dB):
    return pl.pallas_call(
        _tc1b_body,
        grid=(NG,),
        in_specs=[
            pl.BlockSpec((R, D), lambda i: (i, 0)),
            pl.BlockSpec((R, 1), lambda i: (i, 0)),
            pl.BlockSpec((R, 1), lambda i: (i, 0)),
        ],
        out_specs=[
            pl.BlockSpec((R, D), lambda i: (i, 0)),
            pl.BlockSpec((R, 1), lambda i: (i, 0)),
        ],
        out_shape=[
            jax.ShapeDtypeStruct((N, D), jnp.bfloat16),
            jax.ShapeDtypeStruct((N, 1), jnp.float32),
        ],
    )(xww, dA, dB)


def _tc2_body(accp_ref, xw_ref, dinv_ref, b1_ref, w2_ref, w2w_ref,
              as_ref, ad_ref,
              hh_ref, hhw_ref, p_ref, q_ref, pmax_ref):
    i = pl.program_id(0)
    dv = dinv_ref[...]
    acc = accp_ref[0][:, :D] + accp_ref[1][:, :D] + xw_ref[...] * dv
    h1 = jnp.maximum(dv * acc + b1_ref[...], 0.0)
    hh = jnp.dot(h1, w2_ref[...], preferred_element_type=jnp.float32)
    hh_ref[...] = hh
    hhw_ref[...] = jnp.dot(h1, w2w_ref[...],
                           preferred_element_type=jnp.float32
                           ).astype(jnp.bfloat16)
    p = jnp.dot(hh, as_ref[...], preferred_element_type=jnp.float32)
    q = jnp.dot(hh, ad_ref[...], preferred_element_type=jnp.float32)
    p_ref[...] = p
    q_ref[...] = q
    pb = jnp.max(p, axis=(0, 1), keepdims=True)

    @pl.when(i == 0)
    def _():
        pmax_ref[...] = pb

    @pl.when(i > 0)
    def _():
        pmax_ref[...] = jnp.maximum(pmax_ref[...], pb)


def _tc2(accP, xw, dinv, b1, W2, W2w, aS, aD):
    return pl.pallas_call(
        _tc2_body,
        grid=(NG,),
        in_specs=[
            pl.BlockSpec((NC, R, 128), lambda i: (0, i, 0)),
            pl.BlockSpec((R, D), lambda i: (i, 0)),
            pl.BlockSpec((R, 1), lambda i: (i, 0)),
            pl.BlockSpec((1, D), lambda i: (0, 0)),
            pl.BlockSpec((D, D), lambda i: (0, 0)),
            pl.BlockSpec((D, D), lambda i: (0, 0)),
            pl.BlockSpec((D, 1), lambda i: (0, 0)),
            pl.BlockSpec((D, 1), lambda i: (0, 0)),
        ],
        out_specs=[
            pl.BlockSpec((R, D), lambda i: (i, 0)),
            pl.BlockSpec((R, D), lambda i: (i, 0)),
            pl.BlockSpec((R, 1), lambda i: (i, 0)),
            pl.BlockSpec((R, 1), lambda i: (i, 0)),
            pl.BlockSpec((1, 1), lambda i: (0, 0)),
        ],
        out_shape=[
            jax.ShapeDtypeStruct((N, D), jnp.float32),
            jax.ShapeDtypeStruct((N, D), jnp.bfloat16),
            jax.ShapeDtypeStruct((N, 1), jnp.float32),
            jax.ShapeDtypeStruct((N, 1), jnp.float32),
            jax.ShapeDtypeStruct((1, 1), jnp.float32),
        ],
    )(accP, xw, dinv, b1, W2, W2w, aS, aD)


def _tc3_body(accp_ref, hh_ref, p_ref, q_ref, pmax_ref, dinv_ref,
              b2_ref, w3_ref, w3w_ref, g3_ref, g3w_ref):
    p = p_ref[...]
    q = q_ref[...]
    z = p + q
    e_self = jnp.maximum(z, 0.2 * z)
    zu = pmax_ref[0, 0] + q
    u = jnp.maximum(zu, 0.2 * zu)
    w_self = jnp.exp(e_self - u)
    a0 = accp_ref[0]
    a1 = accp_ref[1]
    den = a0[:, D:D + 1] + a1[:, D:D + 1] + w_self + 1e-16
    num = a0[:, :D] + a1[:, :D] + w_self * hh_ref[...]
    h2 = jnp.maximum(num / den + b2_ref[...], 0.0)
    dv = dinv_ref[...]
    g3_ref[...] = jnp.dot(h2, w3_ref[...],
                          preferred_element_type=jnp.float32) * dv
    g3w_ref[...] = (jnp.dot(h2, w3w_ref[...],
                            preferred_element_type=jnp.float32)
                    * dv).astype(jnp.bfloat16)


def _tc3(accP, hh, p, q, pmax, dinv, b2, W3, W3w):
    return pl.pallas_call(
        _tc3_body,
        grid=(NG,),
        in_specs=[
            pl.BlockSpec((NC, R, DP), lambda i: (0, i, 0)),
            pl.BlockSpec((R, D), lambda i: (i, 0)),
            pl.BlockSpec((R, 1), lambda i: (i, 0)),
            pl.BlockSpec((R, 1), lambda i: (i, 0)),
            pl.BlockSpec((1, 1), lambda i: (0, 0)),
            pl.BlockSpec((R, 1), lambda i: (i, 0)),
            pl.BlockSpec((1, D), lambda i: (0, 0)),
            pl.BlockSpec((D, D), lambda i: (0, 0)),
            pl.BlockSpec((D, D), lambda i: (0, 0)),
        ],
        out_specs=[
            pl.BlockSpec((R, D), lambda i: (i, 0)),
            pl.BlockSpec((R, D), lambda i: (i, 0)),
        ],
        out_shape=[
            jax.ShapeDtypeStruct((N, D), jnp.float32),
            jax.ShapeDtypeStruct((N, D), jnp.bfloat16),
        ],
    )(accP, hh, p, q, pmax, dinv, b2, W3, W3w)


def _tc4_body(accp_ref, g3_ref, dinv_ref, b3_ref, wo_ref, bo_ref,
              wb1_ref, bb1_ref, wb2_ref, bb2_ref,
              opt_ref, bt_ref, ge_ref):
    i = pl.program_id(0)
    acc = accp_ref[0][:, :D] + accp_ref[1][:, :D] + g3_ref[...]
    h3 = jnp.maximum(dinv_ref[...] * acc + b3_ref[...], 0.0)
    opt_ref[...] = jnp.dot(h3, wo_ref[...],
                           preferred_element_type=jnp.float32) + bo_ref[...]
    t = jnp.maximum(jnp.dot(h3, wb1_ref[...],
                            preferred_element_type=jnp.float32) + bb1_ref[...],
                    0.0)
    bt_ref[...] = jax.nn.sigmoid(
        jnp.dot(t, wb2_ref[...], preferred_element_type=jnp.float32)
        + bb2_ref[...])
    tot = jnp.sum(h3, axis=0, keepdims=True)

    @pl.when(i == 0)
    def _():
        ge_ref[...] = tot

    @pl.when(i > 0)
    def _():
        ge_ref[...] = ge_ref[...] + tot

    @pl.when(i == NG - 1)
    def _():
        ge_ref[...] = ge_ref[...] * (1.0 / N)


def _tc4(accP, g3, dinv, b3, Wo, bo, Wb1, bb1, Wb2, bb2):
    return pl.pallas_call(
        _tc4_body,
        grid=(NG,),
        in_specs=[
            pl.BlockSpec((NC, R, 128), lambda i: (0, i, 0)),
            pl.BlockSpec((R, D), lambda i: (i, 0)),
            pl.BlockSpec((R, 1), lambda i: (i, 0)),
            pl.BlockSpec((1, D), lambda i: (0, 0)),
            pl.BlockSpec((D, NCLS), lambda i: (0, 0)),
            pl.BlockSpec((1, NCLS), lambda i: (0, 0)),
            pl.BlockSpec((D, 32), lambda i: (0, 0)),
            pl.BlockSpec((1, 32), lambda i: (0, 0)),
            pl.BlockSpec((32, 1), lambda i: (0, 0)),
            pl.BlockSpec((1, 1), lambda i: (0, 0)),
        ],
        out_specs=[
            pl.BlockSpec((R, NCLS), lambda i: (i, 0)),
            pl.BlockSpec((R, 1), lambda i: (i, 0)),
            pl.BlockSpec((1, D), lambda i: (0, 0)),
        ],
        out_shape=[
            jax.ShapeDtypeStruct((N, NCLS), jnp.float32),
            jax.ShapeDtypeStruct((N, 1), jnp.float32),
            jax.ShapeDtypeStruct((1, D), jnp.float32),
        ],
    )(accP, g3, dinv, b3, Wo, bo, Wb1, bb1, Wb2, bb2)


def kernel(x, edge_index, W1, b1, W2, a_src, a_dst, b2, W3, b3, Wo, bo,
           Wb1, bb1, Wb2, bb2):
    src3 = edge_index[0].reshape(NW, NBLK, B)
    dst3 = edge_index[1].reshape(NW, NBLK, B)

    degP = _sc_deg(dst3).reshape(NC, N)
    dA = degP[0].reshape(N, 1)
    dB = degP[1].reshape(N, 1)

    inv_tau = jnp.asarray(_INV_TAU)
    xw, xww = _tc1a(x, W1, W1[:, inv_tau])
    g1w, dinv = _tc1b(xww, dA, dB)
    acc1 = _sc_gcn(g1w, src3, dst3)
    hh, hhw, p, q, pmax = _tc2(acc1, xw, dinv, b1.reshape(1, D),
                               W2, W2[:, inv_tau],
                               a_src.reshape(D, 1), a_dst.reshape(D, 1))

    pmax16 = jnp.broadcast_to(pmax.reshape(1), (16,))
    acc2 = _sc_gat(hhw, p.reshape(N), q.reshape(N), pmax16, src3, dst3)
    g3, g3w = _tc3(acc2, hh, p, q, pmax, dinv,
                   b2.reshape(1, D), W3, W3[:, inv_tau])

    acc3 = _sc_gcn(g3w, src3, dst3)
    opt, bt, ge = _tc4(acc3, g3, dinv, b3.reshape(1, D), Wo,
                       bo.reshape(1, NCLS), Wb1, bb1.reshape(1, 32),
                       Wb2, bb2.reshape(1, 1))
    return opt, bt, ge.reshape(D)
